# baseline (speedup 1.0000x reference)
import functools

import jax
import jax.numpy as jnp
from jax import lax
from jax.experimental import pallas as pl
from jax.experimental.pallas import tpu as pltpu
from jax.experimental.pallas import tpu_sc as plsc

HEADS = 4
OUT_F = 32
NUM_SC = 2
NUM_SUBCORES = 16
NW = NUM_SC * NUM_SUBCORES
CHUNK = 128


def _in_degree_sc(tgt_pad, n_pad):
    ew = tgt_pad.shape[1]
    per_tile = n_pad // NUM_SUBCORES
    mesh = plsc.VectorSubcoreMesh(core_axis_name="c", subcore_axis_name="s")

    @functools.partial(
        pl.kernel,
        out_type=jax.ShapeDtypeStruct((NUM_SC, n_pad), jnp.float32),
        mesh=mesh,
        scratch_types=[
            pltpu.VMEM((ew,), jnp.int32),
            pltpu.VMEM((ew,), jnp.float32),
            pltpu.VMEM((per_tile,), jnp.float32),
            pltpu.VMEM_SHARED((n_pad,), jnp.float32),
        ],
    )
    def deg_kernel(tgt_hbm, out_hbm, idx_v, ones_v, z_v, acc_sh):
        c = lax.axis_index("c")
        s = lax.axis_index("s")
        wid = s * NUM_SC + c

        def fill_ones(j, carry):
            ones_v[pl.ds(j * 16, 16)] = jnp.full((16,), 1.0, jnp.float32)
            return carry

        lax.fori_loop(0, ew // 16, fill_ones, 0)
        for i in range(per_tile // 16):
            z_v[pl.ds(i * 16, 16)] = jnp.zeros((16,), jnp.float32)
        pltpu.sync_copy(z_v, acc_sh.at[pl.ds(s * per_tile, per_tile)])
        pltpu.sync_copy(tgt_hbm.at[wid], idx_v)
        plsc.subcore_barrier()
        pltpu.sync_copy(ones_v, acc_sh.at[idx_v], add=True)
        plsc.subcore_barrier()
        pltpu.sync_copy(
            acc_sh.at[pl.ds(s * per_tile, per_tile)],
            out_hbm.at[c, pl.ds(s * per_tile, per_tile)],
        )

    return deg_kernel(tgt_pad)


def _masked_matmul_tc(x_pad, w, deg_t, blk):
    n_pad, in_f = x_pad.shape
    hf = w.shape[0]

    def mm_kernel(x_ref, w_ref, deg_ref, o_ref):
        y = lax.dot_general(
            x_ref[...], w_ref[...], (((1,), (1,)), ((), ())),
            preferred_element_type=jnp.float32,
        )
        d = deg_ref[:, 0:1] + deg_ref[:, 1:2]
        o_ref[...] = jnp.where(d > 0.0, y, 0.0)

    return pl.pallas_call(
        mm_kernel,
        grid=(n_pad // blk,),
        in_specs=[
            pl.BlockSpec((blk, in_f), lambda i: (i, 0)),
            pl.BlockSpec((hf, in_f), lambda i: (0, 0)),
            pl.BlockSpec((blk, NUM_SC), lambda i: (i, 0)),
        ],
        out_specs=pl.BlockSpec((blk, hf), lambda i: (i, 0)),
        out_shape=jax.ShapeDtypeStruct((n_pad, hf), jnp.float32),
    )(x_pad, w, deg_t)


def kernel(x, edge_index, W, a):
    del a
    n = x.shape[0]
    e = edge_index.shape[1]
    blk = 400
    n_pad = ((n + 255) // 256) * 256
    k = -(-e // (NW * CHUNK))
    e_pad = NW * k * CHUNK

    tgt = edge_index[1].astype(jnp.int32)
    tgt_pad = jnp.concatenate(
        [tgt, jnp.full((e_pad - e,), n_pad - 1, jnp.int32)]
    ).reshape(NW, k * CHUNK)
    deg = _in_degree_sc(tgt_pad, n_pad)
    deg_t = deg.T[:n]
    return _masked_matmul_tc(x, W, deg_t, blk)


# speedup vs baseline: 1.5773x; 1.5773x over previous
import functools

import jax
import jax.numpy as jnp
from jax import lax
from jax.experimental import pallas as pl
from jax.experimental.pallas import tpu as pltpu
from jax.experimental.pallas import tpu_sc as plsc

NUM_SC = 2
NUM_SUBCORES = 16
NW = NUM_SC * NUM_SUBCORES


def _in_degree_sc(edges_flat, e, n_pad):
    ew = e // NW
    ew_buf = ((ew + 15) // 16) * 16
    per_tile = n_pad // NUM_SUBCORES
    mesh = plsc.VectorSubcoreMesh(core_axis_name="c", subcore_axis_name="s")

    @functools.partial(
        pl.kernel,
        out_type=jax.ShapeDtypeStruct((NUM_SC * n_pad,), jnp.float32),
        mesh=mesh,
        scratch_types=[
            pltpu.VMEM((ew,), jnp.int32),
            pltpu.VMEM((ew_buf,), jnp.float32),
            pltpu.VMEM((per_tile,), jnp.float32),
            pltpu.VMEM_SHARED((n_pad,), jnp.float32),
        ],
    )
    def deg_kernel(edges_hbm, out_hbm, idx_v, ones_v, z_v, acc_sh):
        c = lax.axis_index("c")
        s = lax.axis_index("s")
        wid = s * NUM_SC + c
        my_base = pl.multiple_of(s * per_tile, 256)
        out_base = pl.multiple_of(c * n_pad + s * per_tile, 256)

        def fill_ones(j, carry):
            ones_v[pl.ds(j * 16, 16)] = jnp.full((16,), 1.0, jnp.float32)
            return carry

        lax.fori_loop(0, ew_buf // 16, fill_ones, 0)
        for i in range(per_tile // 16):
            z_v[pl.ds(i * 16, 16)] = jnp.zeros((16,), jnp.float32)
        pltpu.sync_copy(z_v, acc_sh.at[pl.ds(my_base, per_tile)])
        edge_base = pl.multiple_of(e + wid * ew, 8)
        pltpu.sync_copy(edges_hbm.at[pl.ds(edge_base, ew)], idx_v)
        plsc.subcore_barrier()
        pltpu.sync_copy(ones_v.at[pl.ds(0, ew)], acc_sh.at[idx_v], add=True)
        plsc.subcore_barrier()
        pltpu.sync_copy(
            acc_sh.at[pl.ds(my_base, per_tile)],
            out_hbm.at[pl.ds(out_base, per_tile)],
        )

    return deg_kernel(edges_flat).reshape(NUM_SC, n_pad)


def _masked_matmul_tc(x, w, deg_t, blk):
    n, in_f = x.shape
    hf = w.shape[0]

    def mm_kernel(x_ref, w_ref, deg_ref, o_ref):
        y = lax.dot_general(
            x_ref[...], w_ref[...], (((1,), (1,)), ((), ())),
            preferred_element_type=jnp.float32,
        )
        d = deg_ref[:, 0:1] + deg_ref[:, 1:2]
        o_ref[...] = jnp.where(d > 0, y, 0.0)

    return pl.pallas_call(
        mm_kernel,
        grid=(n // blk,),
        in_specs=[
            pl.BlockSpec((blk, in_f), lambda i: (i, 0)),
            pl.BlockSpec((hf, in_f), lambda i: (0, 0)),
            pl.BlockSpec((blk, NUM_SC), lambda i: (i, 0)),
        ],
        out_specs=pl.BlockSpec((blk, hf), lambda i: (i, 0)),
        out_shape=jax.ShapeDtypeStruct((n, hf), jnp.float32),
    )(x, w, deg_t)


def kernel(x, edge_index, W, a):
    del a
    n = x.shape[0]
    blk = 2000
    n_pad = ((n + 255) // 256) * 256
    e = edge_index.shape[1]
    deg = _in_degree_sc(edge_index.reshape(-1), e, n_pad)
    deg_t = deg.T
    return _masked_matmul_tc(x, W, deg_t, blk)
